# baseline (speedup 1.0000x reference)
import functools

import jax
import jax.numpy as jnp
from jax import lax
from jax.experimental import pallas as pl
from jax.experimental.pallas import tpu as pltpu
from jax.experimental.pallas import tpu_sc as plsc

N = 10000
E = 320000
D = 128
K = 3

NC, NS = 2, 16
NW = NC * NS
CH = 128
STEPS = 80
EPAD = NW * STEPS * CH
NPAD = 10112
ROWS_T = NPAD // NS
DEGW = 16

_sc_cache = {}


def _sc_kernels():
    if "k" in _sc_cache:
        return _sc_cache["k"]

    mesh = plsc.VectorSubcoreMesh(
        core_axis_name="c", subcore_axis_name="s",
        num_cores=NC, num_subcores=NS)

    GS = 8
    S_FAST, S_SLOW = 128, 32

    @functools.partial(
        pl.kernel,
        out_type=jax.ShapeDtypeStruct((NC, NPAD, D), jnp.float32),
        mesh=mesh,
        scratch_types=[
            pltpu.VMEM((2, GS, CH), jnp.int32),
            pltpu.VMEM((2, GS, CH), jnp.int32),
            pltpu.VMEM((CH, D), jnp.float32),
            pltpu.VMEM((CH, D), jnp.float32),
            pltpu.VMEM_SHARED((NPAD, D), jnp.float32),
            pltpu.SemaphoreType.DMA,
            pltpu.SemaphoreType.DMA,
            pltpu.SemaphoreType.DMA,
            pltpu.SemaphoreType.DMA,
        ],
    )
    def hop_kernel(src_hbm, dst_hbm, u_hbm, out_hbm,
                   sidx, didx, rows0, rows1, acc, isem0, isem1, gsem0, gsem1):
        c = lax.axis_index("c")
        s = lax.axis_index("s")
        rows = (rows0, rows1)
        gsems = (gsem0, gsem1)
        isems = (isem0, isem1)

        def fillz(i, carry):
            for k in range(D // 16):
                rows0[i, pl.ds(k * 16, 16)] = jnp.zeros((16,), jnp.float32)
            return carry

        lax.fori_loop(0, CH, fillz, 0)
        for k in range(ROWS_T // CH):
            pltpu.sync_copy(rows0, acc.at[pl.ds(s * ROWS_T + k * CH, CH)])
        rem = ROWS_T % CH
        if rem:
            pltpu.sync_copy(
                rows0.at[pl.ds(0, rem)],
                acc.at[pl.ds(s * ROWS_T + (ROWS_T // CH) * CH, rem)])

        def pfg(base, g, slot):
            off = pl.multiple_of(base + g * GS, GS)
            pltpu.async_copy(src_hbm.at[pl.ds(off, GS)], sidx.at[slot],
                             isems[slot])
            pltpu.async_copy(dst_hbm.at[pl.ds(off, GS)], didx.at[slot],
                             isems[slot])

        def pfg_wait(base, g, slot):
            off = pl.multiple_of(base + g * GS, GS)
            pltpu.make_async_copy(src_hbm.at[pl.ds(off, GS)], sidx.at[slot],
                                  isems[slot]).wait()
            pltpu.make_async_copy(src_hbm.at[pl.ds(off, GS)], didx.at[slot],
                                  isems[slot]).wait()

        def g_start(slot, row, par):
            pltpu.async_copy(u_hbm.at[sidx.at[slot, row]], rows[par],
                             gsems[par])

        def g_wait(slot, row, par):
            pltpu.make_async_copy(u_hbm.at[sidx.at[slot, row]], rows[par],
                                  gsems[par]).wait()

        def scat(slot, row, par):
            pltpu.sync_copy(rows[par], acc.at[didx.at[slot, row]], add=True)

        def run(base, ngroups):
            def group(g, slot, refill, wait_next, last_start):
                for p in range(GS):
                    if p == GS - 2 and wait_next:
                        pfg_wait(base, g + 1, slot ^ 1)
                    if p < GS - 1:
                        g_start(slot, p + 1, (p + 1) % 2)
                    elif last_start:
                        g_start(slot ^ 1, 0, 0)
                    g_wait(slot, p, p % 2)
                    scat(slot, p, p % 2)
                if refill:
                    pfg(base, g + 2, slot)

            pfg(base, 0, 0)
            pfg(base, 1, 1)
            pfg_wait(base, 0, 0)
            g_start(0, 0, 0)

            def super_group(i, carry):
                g = 2 * i
                group(g, 0, True, True, True)
                group(g + 1, 1, True, True, True)
                return carry

            lax.fori_loop(0, ngroups // 2 - 1, super_group, 0)
            group(ngroups - 2, 0, False, True, True)
            group(ngroups - 1, 1, False, False, False)

        plsc.subcore_barrier()

        @pl.when(c == 0)
        def _fast():
            run(s * (S_FAST + S_SLOW), S_FAST // GS)

        @pl.when(c == 1)
        def _slow():
            run(s * (S_FAST + S_SLOW) + S_FAST, S_SLOW // GS)

        plsc.subcore_barrier()
        pltpu.sync_copy(acc.at[pl.ds(s * ROWS_T, ROWS_T)],
                        out_hbm.at[c].at[pl.ds(s * ROWS_T, ROWS_T)])

    _sc_cache["k"] = hop_kernel
    return _sc_cache["k"]


RB = 1000


def _prep_body(degp_ref, x_ref, u0_ref, dinv_ref, sq_ref):
    deg = degp_ref[0, :, :DEGW] + degp_ref[1, :, :DEGW] + 1.0
    di = lax.rsqrt(deg)
    u0_ref[...] = x_ref[...] * di[:, :1]
    dinv_ref[...] = 1.0 / deg
    sq_ref[...] = deg * di


def _prep(degp, x):
    return pl.pallas_call(
        _prep_body,
        grid=(N // RB,),
        in_specs=[
            pl.BlockSpec((NC, RB, D), lambda i: (0, i, 0)),
            pl.BlockSpec((RB, D), lambda i: (i, 0)),
        ],
        out_specs=[
            pl.BlockSpec((RB, D), lambda i: (i, 0)),
            pl.BlockSpec((RB, DEGW), lambda i: (i, 0)),
            pl.BlockSpec((RB, DEGW), lambda i: (i, 0)),
        ],
        out_shape=[
            jax.ShapeDtypeStruct((N, D), jnp.float32),
            jax.ShapeDtypeStruct((N, DEGW), jnp.float32),
            jax.ShapeDtypeStruct((N, DEGW), jnp.float32),
        ],
    )(degp, x)


def _combine_body(p_ref, u_ref, dinv_ref, out_ref):
    out_ref[...] = (p_ref[0] + p_ref[1] + u_ref[...]) * dinv_ref[:, :1]


def _combine(p, u, dinv):
    return pl.pallas_call(
        _combine_body,
        grid=(N // RB,),
        in_specs=[
            pl.BlockSpec((NC, RB, D), lambda i: (0, i, 0)),
            pl.BlockSpec((RB, D), lambda i: (i, 0)),
            pl.BlockSpec((RB, DEGW), lambda i: (i, 0)),
        ],
        out_specs=pl.BlockSpec((RB, D), lambda i: (i, 0)),
        out_shape=jax.ShapeDtypeStruct((N, D), jnp.float32),
    )(p, u, dinv)


def _final_body(u_ref, sq_ref, w_ref, b_ref, out_ref):
    h = u_ref[...] * sq_ref[:, :1]
    out_ref[...] = (
        jnp.dot(h, w_ref[...], preferred_element_type=jnp.float32) + b_ref[...]
    )


def _final(u, sq, W, b2):
    return pl.pallas_call(
        _final_body,
        grid=(N // RB,),
        in_specs=[
            pl.BlockSpec((RB, D), lambda i: (i, 0)),
            pl.BlockSpec((RB, DEGW), lambda i: (i, 0)),
            pl.BlockSpec((D, D), lambda i: (0, 0)),
            pl.BlockSpec((1, D), lambda i: (0, 0)),
        ],
        out_specs=pl.BlockSpec((RB, D), lambda i: (i, 0)),
        out_shape=jax.ShapeDtypeStruct((N, D), jnp.float32),
    )(u, sq, W, b2)


def kernel(x, edge_index, W, b):
    npad_e = EPAD - E
    src = jnp.concatenate(
        [edge_index[0], jnp.zeros((npad_e,), jnp.int32)]).reshape(
            NW * STEPS, CH)
    dst = jnp.concatenate(
        [edge_index[1],
         N + (jnp.arange(npad_e, dtype=jnp.int32) % (NPAD - N))]).reshape(
             NW * STEPS, CH)
    hop_kernel = _sc_kernels()
    degp = hop_kernel(src, dst, jnp.ones((N, D), jnp.float32))
    u, dinv, sq = _prep(degp, x)
    for _ in range(K):
        p = hop_kernel(src, dst, u)
        u = _combine(p, u, dinv)
    return _final(u, sq, W, b.reshape(1, D))
